# baseline (device time: 34813 ns/iter reference)
import jax
import jax.numpy as jnp
from jax import lax
from jax.experimental import pallas as pl
from jax.experimental.pallas import tpu as pltpu

B = 4
S_HALF = 256
H = 8
D = 64
N = 1024


def kernel(O, Wo):
    w3 = Wo.reshape(H, D, N)

    def body(o_ref, w_ref, out_hbm, send_buf, recv_buf, res_buf,
             send_sems, recv_sems, out_sems):
        my_x = lax.axis_index("x")
        my_y = lax.axis_index("y")
        peer_y = 1 - my_y

        barrier = pltpu.get_barrier_semaphore()
        pl.semaphore_signal(
            barrier, inc=1,
            device_id=(my_x, peer_y), device_id_type=pl.DeviceIdType.MESH,
        )
        pl.semaphore_wait(barrier, 1)

        def chunk_matmul(c, s0):
            acc = None
            for h in range(H):
                a = o_ref[c, pl.ds(s0, S_HALF), h, :].astype(jnp.bfloat16)
                wh = w_ref[h, :, :].astype(jnp.bfloat16)
                p = jnp.dot(a, wh, preferred_element_type=jnp.float32)
                acc = p if acc is None else acc + p
            return acc

        rdmas = []
        for c in range(B):
            send_buf[c, :, :] = chunk_matmul(c, peer_y * S_HALF).astype(
                jnp.bfloat16
            )
            r = pltpu.make_async_remote_copy(
                src_ref=send_buf.at[c],
                dst_ref=recv_buf.at[c],
                send_sem=send_sems.at[c],
                recv_sem=recv_sems.at[c],
                device_id=(my_x, peer_y),
                device_id_type=pl.DeviceIdType.MESH,
            )
            r.start()
            rdmas.append(r)

        for c in range(B):
            res_buf[c, :, :] = chunk_matmul(c, my_y * S_HALF)

        copies = []
        for c in range(B):
            rdmas[c].wait_recv()
            res_buf[c, :, :] = res_buf[c, :, :] + recv_buf[c, :, :].astype(
                jnp.float32
            )
            cp = pltpu.make_async_copy(res_buf.at[c], out_hbm.at[c],
                                       out_sems.at[c])
            cp.start()
            copies.append(cp)
        for cp in copies:
            cp.wait()
        for r in rdmas:
            r.wait_send()

    return pl.pallas_call(
        body,
        out_shape=jax.ShapeDtypeStruct((B, S_HALF, N), jnp.float32),
        in_specs=[
            pl.BlockSpec(memory_space=pltpu.VMEM),
            pl.BlockSpec(memory_space=pltpu.VMEM),
        ],
        out_specs=pl.BlockSpec(memory_space=pl.ANY),
        scratch_shapes=[
            pltpu.VMEM((B, S_HALF, N), jnp.bfloat16),
            pltpu.VMEM((B, S_HALF, N), jnp.bfloat16),
            pltpu.VMEM((B, S_HALF, N), jnp.float32),
            pltpu.SemaphoreType.DMA((B,)),
            pltpu.SemaphoreType.DMA((B,)),
            pltpu.SemaphoreType.DMA((B,)),
        ],
        compiler_params=pltpu.CompilerParams(collective_id=0),
    )(O, w3)


# device time: 32313 ns/iter; 1.0774x vs baseline; 1.0774x over previous
import jax
import jax.numpy as jnp
from jax import lax
from jax.experimental import pallas as pl
from jax.experimental.pallas import tpu as pltpu

B = 4
S_HALF = 256
H = 8
D = 64
N = 1024
NH = 512


def kernel(O, Wo):
    w3 = Wo.reshape(H, D, N)

    def body(o_hbm, w_ref, out_hbm, o_vmem, send_y, recv_y, send_x, recv_x,
             res, resx, o_sems, ysend_sems, yrecv_sems, xsend_sems,
             xrecv_sems, out_sems):
        my_x = lax.axis_index("x")
        my_y = lax.axis_index("y")
        peer_y = 1 - my_y
        peer_x = 1 - my_x
        col0 = my_x * NH

        o_dmas = []
        for c in range(B):
            d = pltpu.make_async_copy(o_hbm.at[c], o_vmem.at[c],
                                      o_sems.at[c])
            d.start()
            o_dmas.append(d)

        barrier = pltpu.get_barrier_semaphore()
        for dev in ((my_x, peer_y), (peer_x, my_y)):
            pl.semaphore_signal(
                barrier, inc=1,
                device_id=dev, device_id_type=pl.DeviceIdType.MESH,
            )
        pl.semaphore_wait(barrier, 2)

        def chunk_matmul(c, s0):
            acc = None
            for h in range(H):
                a = o_vmem[c, pl.ds(s0, S_HALF), h, :].astype(jnp.bfloat16)
                wh = w_ref[h, :, pl.ds(col0, NH)].astype(jnp.bfloat16)
                p = jnp.dot(a, wh, preferred_element_type=jnp.float32)
                acc = p if acc is None else acc + p
            return acc

        y_rdmas = []
        for c in range(B):
            o_dmas[c].wait()
            send_y[c, :, :] = chunk_matmul(c, peer_y * S_HALF).astype(
                jnp.bfloat16
            )
            r = pltpu.make_async_remote_copy(
                src_ref=send_y.at[c],
                dst_ref=recv_y.at[c],
                send_sem=ysend_sems.at[c],
                recv_sem=yrecv_sems.at[c],
                device_id=(my_x, peer_y),
                device_id_type=pl.DeviceIdType.MESH,
            )
            r.start()
            y_rdmas.append(r)

        for c in range(B):
            res[c, :, :] = chunk_matmul(c, my_y * S_HALF)

        x_rdmas = []
        out_dmas = []
        for c in range(B):
            y_rdmas[c].wait_recv()
            res[c, :, :] = res[c, :, :] + recv_y[c, :, :].astype(jnp.float32)
            send_x[c, :, :] = res[c, :, :].astype(jnp.bfloat16)
            rx = pltpu.make_async_remote_copy(
                src_ref=send_x.at[c],
                dst_ref=recv_x.at[c],
                send_sem=xsend_sems.at[c],
                recv_sem=xrecv_sems.at[c],
                device_id=(peer_x, my_y),
                device_id_type=pl.DeviceIdType.MESH,
            )
            rx.start()
            x_rdmas.append(rx)
            od = pltpu.make_async_copy(
                res.at[c], out_hbm.at[c, :, pl.ds(col0, NH)],
                out_sems.at[c, 0],
            )
            od.start()
            out_dmas.append(od)

        for c in range(B):
            x_rdmas[c].wait_recv()
            resx[c, :, :] = recv_x[c, :, :].astype(jnp.float32)
            od = pltpu.make_async_copy(
                resx.at[c], out_hbm.at[c, :, pl.ds(peer_x * NH, NH)],
                out_sems.at[c, 1],
            )
            od.start()
            out_dmas.append(od)

        for od in out_dmas:
            od.wait()
        for r in y_rdmas:
            r.wait_send()
        for r in x_rdmas:
            r.wait_send()

    return pl.pallas_call(
        body,
        out_shape=jax.ShapeDtypeStruct((B, S_HALF, N), jnp.float32),
        in_specs=[
            pl.BlockSpec(memory_space=pl.ANY),
            pl.BlockSpec(memory_space=pltpu.VMEM),
        ],
        out_specs=pl.BlockSpec(memory_space=pl.ANY),
        scratch_shapes=[
            pltpu.VMEM((B, 2 * S_HALF, H, D), jnp.float32),
            pltpu.VMEM((B, S_HALF, NH), jnp.bfloat16),
            pltpu.VMEM((B, S_HALF, NH), jnp.bfloat16),
            pltpu.VMEM((B, S_HALF, NH), jnp.bfloat16),
            pltpu.VMEM((B, S_HALF, NH), jnp.bfloat16),
            pltpu.VMEM((B, S_HALF, NH), jnp.float32),
            pltpu.VMEM((B, S_HALF, NH), jnp.float32),
            pltpu.SemaphoreType.DMA((B,)),
            pltpu.SemaphoreType.DMA((B,)),
            pltpu.SemaphoreType.DMA((B,)),
            pltpu.SemaphoreType.DMA((B,)),
            pltpu.SemaphoreType.DMA((B,)),
            pltpu.SemaphoreType.DMA((B, 2)),
        ],
        compiler_params=pltpu.CompilerParams(collective_id=0),
    )(O, w3)


# device time: 28712 ns/iter; 1.2125x vs baseline; 1.1254x over previous
import jax
import jax.numpy as jnp
from jax import lax
from jax.experimental import pallas as pl
from jax.experimental.pallas import tpu as pltpu

B = 4
S_HALF = 256
H = 8
D = 64
K = H * D
N = 1024
NH = 512


def kernel(O, Wo):
    def body(o_hbm, w_ref, out_hbm, o_vmem, send_y, recv_y, send_x, recv_x,
             res, resx, o_sems, ysend_sems, yrecv_sems, xsend_sems,
             xrecv_sems, out_sems):
        my_x = lax.axis_index("x")
        my_y = lax.axis_index("y")
        peer_y = 1 - my_y
        peer_x = 1 - my_x
        col0 = my_x * NH

        o_dmas = []
        for c in range(B):
            d = pltpu.make_async_copy(o_hbm.at[c], o_vmem.at[c],
                                      o_sems.at[c])
            d.start()
            o_dmas.append(d)

        barrier = pltpu.get_barrier_semaphore()
        for dev in ((my_x, peer_y), (peer_x, my_y)):
            pl.semaphore_signal(
                barrier, inc=1,
                device_id=dev, device_id_type=pl.DeviceIdType.MESH,
            )
        pl.semaphore_wait(barrier, 2)

        w_my = w_ref[:, pl.ds(col0, NH)].astype(jnp.bfloat16)

        def chunk_matmul(c, s0):
            a = o_vmem[c, pl.ds(s0, S_HALF), :, :].reshape(S_HALF, K)
            return jnp.dot(a.astype(jnp.bfloat16), w_my,
                           preferred_element_type=jnp.float32)

        y_rdmas = []
        for c in range(B):
            o_dmas[c].wait()
            send_y[c, :, :] = chunk_matmul(c, peer_y * S_HALF).astype(
                jnp.bfloat16
            )
            r = pltpu.make_async_remote_copy(
                src_ref=send_y.at[c],
                dst_ref=recv_y.at[c],
                send_sem=ysend_sems.at[c],
                recv_sem=yrecv_sems.at[c],
                device_id=(my_x, peer_y),
                device_id_type=pl.DeviceIdType.MESH,
            )
            r.start()
            y_rdmas.append(r)

        x_rdmas = []
        out_dmas = []
        for c in range(B):
            res[c, :, :] = chunk_matmul(c, my_y * S_HALF)
            y_rdmas[c].wait_recv()
            res[c, :, :] = res[c, :, :] + recv_y[c, :, :].astype(jnp.float32)
            send_x[c, :, :] = res[c, :, :].astype(jnp.bfloat16)
            rx = pltpu.make_async_remote_copy(
                src_ref=send_x.at[c],
                dst_ref=recv_x.at[c],
                send_sem=xsend_sems.at[c],
                recv_sem=xrecv_sems.at[c],
                device_id=(peer_x, my_y),
                device_id_type=pl.DeviceIdType.MESH,
            )
            rx.start()
            x_rdmas.append(rx)
            od = pltpu.make_async_copy(
                res.at[c], out_hbm.at[c, :, pl.ds(col0, NH)],
                out_sems.at[c, 0],
            )
            od.start()
            out_dmas.append(od)

        for c in range(B):
            x_rdmas[c].wait_recv()
            resx[c, :, :] = recv_x[c, :, :].astype(jnp.float32)
            od = pltpu.make_async_copy(
                resx.at[c], out_hbm.at[c, :, pl.ds(peer_x * NH, NH)],
                out_sems.at[c, 1],
            )
            od.start()
            out_dmas.append(od)

        for od in out_dmas:
            od.wait()
        for r in y_rdmas:
            r.wait_send()
        for r in x_rdmas:
            r.wait_send()

    return pl.pallas_call(
        body,
        out_shape=jax.ShapeDtypeStruct((B, S_HALF, N), jnp.float32),
        in_specs=[
            pl.BlockSpec(memory_space=pltpu.MemorySpace.HBM),
            pl.BlockSpec(memory_space=pltpu.VMEM),
        ],
        out_specs=pl.BlockSpec(memory_space=pltpu.MemorySpace.HBM),
        scratch_shapes=[
            pltpu.VMEM((B, 2 * S_HALF, H, D), jnp.float32),
            pltpu.VMEM((B, S_HALF, NH), jnp.bfloat16),
            pltpu.VMEM((B, S_HALF, NH), jnp.bfloat16),
            pltpu.VMEM((B, S_HALF, NH), jnp.bfloat16),
            pltpu.VMEM((B, S_HALF, NH), jnp.bfloat16),
            pltpu.VMEM((B, S_HALF, NH), jnp.float32),
            pltpu.VMEM((B, S_HALF, NH), jnp.float32),
            pltpu.SemaphoreType.DMA((B,)),
            pltpu.SemaphoreType.DMA((B,)),
            pltpu.SemaphoreType.DMA((B,)),
            pltpu.SemaphoreType.DMA((B,)),
            pltpu.SemaphoreType.DMA((B,)),
            pltpu.SemaphoreType.DMA((B, 2)),
        ],
        compiler_params=pltpu.CompilerParams(collective_id=0),
    )(O, Wo)


# device time: 26611 ns/iter; 1.3082x vs baseline; 1.0790x over previous
import jax
import jax.numpy as jnp
from jax import lax
from jax.experimental import pallas as pl
from jax.experimental.pallas import tpu as pltpu

B = 4
S_HALF = 256
H = 8
D = 64
K = H * D
N = 1024
NH = 512


def kernel(O, Wo):
    ot = jnp.transpose(O, (0, 2, 3, 1))

    def body(ot_ref, w_ref, out_hbm, send_y, recv_y, send_x, recv_x,
             res, resx, ysend_sems, yrecv_sems, xsend_sems, xrecv_sems,
             out_sems):
        my_x = lax.axis_index("x")
        my_y = lax.axis_index("y")
        peer_y = 1 - my_y
        peer_x = 1 - my_x
        col0 = my_x * NH

        barrier = pltpu.get_barrier_semaphore()
        for dev in ((my_x, peer_y), (peer_x, my_y)):
            pl.semaphore_signal(
                barrier, inc=1,
                device_id=dev, device_id_type=pl.DeviceIdType.MESH,
            )
        pl.semaphore_wait(barrier, 2)

        w_my = w_ref[:, pl.ds(col0, NH)].astype(jnp.bfloat16)

        def chunk_matmul(c, s0):
            lhsT = ot_ref[c, :, :, pl.ds(s0, S_HALF)].reshape(K, S_HALF)
            return lax.dot_general(
                lhsT.astype(jnp.bfloat16), w_my,
                (((0,), (0,)), ((), ())),
                preferred_element_type=jnp.float32,
            )

        y_rdmas = []
        for c in range(B):
            send_y[c, :, :] = chunk_matmul(c, peer_y * S_HALF).astype(
                jnp.bfloat16
            )
            r = pltpu.make_async_remote_copy(
                src_ref=send_y.at[c],
                dst_ref=recv_y.at[c],
                send_sem=ysend_sems.at[c],
                recv_sem=yrecv_sems.at[c],
                device_id=(my_x, peer_y),
                device_id_type=pl.DeviceIdType.MESH,
            )
            r.start()
            y_rdmas.append(r)

        x_rdmas = []
        out_dmas = []
        for c in range(B):
            res[c, :, :] = chunk_matmul(c, my_y * S_HALF)
            y_rdmas[c].wait_recv()
            res[c, :, :] = res[c, :, :] + recv_y[c, :, :].astype(jnp.float32)
            send_x[c, :, :] = res[c, :, :].astype(jnp.bfloat16)
            rx = pltpu.make_async_remote_copy(
                src_ref=send_x.at[c],
                dst_ref=recv_x.at[c],
                send_sem=xsend_sems.at[c],
                recv_sem=xrecv_sems.at[c],
                device_id=(peer_x, my_y),
                device_id_type=pl.DeviceIdType.MESH,
            )
            rx.start()
            x_rdmas.append(rx)
            od = pltpu.make_async_copy(
                res.at[c], out_hbm.at[c, :, pl.ds(col0, NH)],
                out_sems.at[c, 0],
            )
            od.start()
            out_dmas.append(od)

        for c in range(B):
            x_rdmas[c].wait_recv()
            resx[c, :, :] = recv_x[c, :, :].astype(jnp.float32)
            od = pltpu.make_async_copy(
                resx.at[c], out_hbm.at[c, :, pl.ds(peer_x * NH, NH)],
                out_sems.at[c, 1],
            )
            od.start()
            out_dmas.append(od)

        for od in out_dmas:
            od.wait()
        for r in y_rdmas:
            r.wait_send()
        for r in x_rdmas:
            r.wait_send()

    return pl.pallas_call(
        body,
        out_shape=jax.ShapeDtypeStruct((B, S_HALF, N), jnp.float32),
        in_specs=[
            pl.BlockSpec(memory_space=pltpu.VMEM),
            pl.BlockSpec(memory_space=pltpu.VMEM),
        ],
        out_specs=pl.BlockSpec(memory_space=pltpu.MemorySpace.HBM),
        scratch_shapes=[
            pltpu.VMEM((B, S_HALF, NH), jnp.bfloat16),
            pltpu.VMEM((B, S_HALF, NH), jnp.bfloat16),
            pltpu.VMEM((B, S_HALF, NH), jnp.bfloat16),
            pltpu.VMEM((B, S_HALF, NH), jnp.bfloat16),
            pltpu.VMEM((B, S_HALF, NH), jnp.float32),
            pltpu.VMEM((B, S_HALF, NH), jnp.float32),
            pltpu.SemaphoreType.DMA((B,)),
            pltpu.SemaphoreType.DMA((B,)),
            pltpu.SemaphoreType.DMA((B,)),
            pltpu.SemaphoreType.DMA((B,)),
            pltpu.SemaphoreType.DMA((B, 2)),
        ],
        compiler_params=pltpu.CompilerParams(collective_id=0),
    )(ot, Wo)
